# baseline (speedup 1.0000x reference)
import jax
import jax.numpy as jnp
from jax import lax
from jax.experimental import pallas as pl
from jax.experimental.pallas import tpu as pltpu
from jax.experimental.pallas import tpu_sc as plsc

N = 10000
E = 320000
D_IN = 128
D_H = 64
D_OUT = 7
D_OUT_PAD = 16

NC = 2
NS = 16
NW = NC * NS
CHUNK = 112
KSTEPS = 92
EPW = KSTEPS * CHUNK
E_PAD = NW * EPW
N_PAD = 10240
STRIPE = N_PAD // NS

_MESH = plsc.VectorSubcoreMesh(core_axis_name="c", subcore_axis_name="s")
_SC_PARAMS = pltpu.CompilerParams(use_tc_tiling_on_sc=False)


def _deg_body(dst3, zeros, out, idx_v, ones_v, deg_sh):
    c = lax.axis_index("c")
    s = lax.axis_index("s")
    wid = s * NC + c

    @pl.loop(0, CHUNK)
    def _(i):
        ones_v[i, :] = jnp.ones((16,), jnp.float32)

    pltpu.sync_copy(zeros.at[pl.ds(s * STRIPE, STRIPE)],
                    deg_sh.at[pl.ds(s * STRIPE, STRIPE)])
    plsc.subcore_barrier()

    pltpu.sync_copy(dst3.at[wid], idx_v)

    @pl.loop(0, KSTEPS)
    def _(j):
        pltpu.sync_copy(ones_v, deg_sh.at[idx_v.at[j]], add=True)

    plsc.subcore_barrier()
    pltpu.sync_copy(deg_sh.at[pl.ds(s * STRIPE, STRIPE)],
                    out.at[pl.ds(c * N_PAD + s * STRIPE, STRIPE)])


def _deg_counts(dst3, zeros16):
    k = pl.kernel(
        _deg_body,
        out_type=jax.ShapeDtypeStruct((NC * N_PAD, 16), jnp.float32),
        mesh=_MESH,
        scratch_types=[
            pltpu.VMEM((KSTEPS, CHUNK), jnp.int32),
            pltpu.VMEM((CHUNK, 16), jnp.float32),
            pltpu.VMEM_SHARED((N_PAD, 16), jnp.float32),
        ],
        compiler_params=_SC_PARAMS,
    )
    return k(dst3, zeros16)


_DEPTH = 4
_LEAD = 2


def _agg_body(u_hbm, src3, dst3, zeros, out,
              isrc, idst, r0, r1, r2, r3, agg_sh,
              g0, g1, g2, g3, s0, s1, s2, s3):
    rows = [r0, r1, r2, r3]
    gsem = [g0, g1, g2, g3]
    ssem = [s0, s1, s2, s3]
    c = lax.axis_index("c")
    s = lax.axis_index("s")
    wid = s * NC + c

    pltpu.sync_copy(zeros.at[pl.ds(s * STRIPE, STRIPE)],
                    agg_sh.at[pl.ds(s * STRIPE, STRIPE)])
    plsc.subcore_barrier()

    pltpu.sync_copy(src3.at[wid], isrc)
    pltpu.sync_copy(dst3.at[wid], idst)

    def wait_gather(b):
        pltpu.make_async_copy(u_hbm.at[isrc.at[0]], rows[b], gsem[b]).wait()

    def wait_scatter(b):
        pltpu.make_async_copy(rows[b], agg_sh.at[idst.at[0]], ssem[b]).wait()

    for i in range(_LEAD):
        pltpu.async_copy(u_hbm.at[isrc.at[i]], rows[i], gsem[i])

    @pl.loop(0, KSTEPS, step=_DEPTH)
    def _(j):
        for o in range(_DEPTH):
            i = j + o
            b = o
            bn = (o + _LEAD) % _DEPTH
            wait_gather(b)
            pltpu.async_copy(rows[b], agg_sh.at[idst.at[i]], ssem[b],
                             add=True)

            @pl.when(i + _LEAD < KSTEPS)
            def _():
                @pl.when(i >= _DEPTH - _LEAD)
                def _():
                    wait_scatter(bn)
                pltpu.async_copy(u_hbm.at[isrc.at[i + _LEAD]], rows[bn],
                                 gsem[bn])

    for b in range(_DEPTH):
        wait_scatter(b)

    plsc.subcore_barrier()
    pltpu.sync_copy(agg_sh.at[pl.ds(s * STRIPE, STRIPE)],
                    out.at[pl.ds(c * N_PAD + s * STRIPE, STRIPE)])


def _edge_aggregate(u, src3, dst3, zeros, width):
    k = pl.kernel(
        _agg_body,
        out_type=jax.ShapeDtypeStruct((NC * N_PAD, width), jnp.float32),
        mesh=_MESH,
        scratch_types=(
            [pltpu.VMEM((KSTEPS, CHUNK), jnp.int32)] * 2
            + [pltpu.VMEM((CHUNK, width), jnp.float32)] * _DEPTH
            + [pltpu.VMEM_SHARED((N_PAD, width), jnp.float32)]
            + [pltpu.SemaphoreType.DMA] * (2 * _DEPTH)
        ),
        compiler_params=_SC_PARAMS,
    )
    return k(u, src3, dst3, zeros)


def _dinv(deg_ref):
    d = deg_ref[0:N, 0:1] + deg_ref[N_PAD:N_PAD + N, 0:1] + 1.0
    return lax.rsqrt(d)


def _u1_body(x_ref, w1_ref, deg_ref, u1_ref):
    h = jnp.dot(x_ref[...], w1_ref[...], preferred_element_type=jnp.float32)
    u1_ref[...] = _dinv(deg_ref) * h


def _u2_body(agg_ref, u1_ref, deg_ref, b1_ref, w2_ref, u2_ref):
    dinv = _dinv(deg_ref)
    agg = agg_ref[0:N, :] + agg_ref[N_PAD:N_PAD + N, :]
    h1 = jnp.maximum(dinv * (agg + u1_ref[...]) + b1_ref[...], 0.0)
    u2_ref[...] = dinv * jnp.dot(h1, w2_ref[...],
                                 preferred_element_type=jnp.float32)


def _out_body(agg_ref, u2_ref, deg_ref, b2_ref, o_ref):
    dinv = _dinv(deg_ref)
    agg = agg_ref[0:N, :] + agg_ref[N_PAD:N_PAD + N, :]
    o_ref[...] = dinv * (agg + u2_ref[...]) + b2_ref[...]


def _tc_call(body, out_shape):
    return pl.pallas_call(
        body,
        out_shape=jax.ShapeDtypeStruct(out_shape, jnp.float32),
    )


@jax.jit
def kernel(x, edge_index, W1, b1, W2, b2):
    n_pad_e = EPW - E // NW
    src2 = edge_index[0].astype(jnp.int32).reshape(NW, E // NW)
    dst2 = edge_index[1].astype(jnp.int32).reshape(NW, E // NW)
    if n_pad_e:
        pad_src = jnp.zeros((NW, n_pad_e), jnp.int32)
        pad_dst = jnp.broadcast_to(
            N + (jnp.arange(n_pad_e, dtype=jnp.int32) % (N_PAD - N)),
            (NW, n_pad_e))
        src2 = jnp.concatenate([src2, pad_src], axis=1)
        dst2 = jnp.concatenate([dst2, pad_dst], axis=1)
    src3 = src2.reshape(NW, KSTEPS, CHUNK)
    dst3 = dst2.reshape(NW, KSTEPS, CHUNK)
    z16 = jnp.zeros((N_PAD, 16), jnp.float32)
    z64 = jnp.zeros((N_PAD, D_H), jnp.float32)
    w2p = jnp.pad(W2, ((0, 0), (0, D_OUT_PAD - D_OUT)))
    b1r = b1.reshape(1, D_H)
    b2r = jnp.pad(b2, (0, D_OUT_PAD - D_OUT)).reshape(1, D_OUT_PAD)

    deg = _deg_counts(dst3, z16)

    u1 = _tc_call(_u1_body, (N, D_H))(x, W1, deg)
    agg1 = _edge_aggregate(u1, src3, dst3, z64, D_H)
    u2 = _tc_call(_u2_body, (N, D_OUT_PAD))(agg1, u1, deg, b1r, w2p)
    agg2 = _edge_aggregate(u2, src3, dst3, z16, D_OUT_PAD)
    out = _tc_call(_out_body, (N, D_OUT_PAD))(agg2, u2, deg, b2r)
    return out[:, :D_OUT]


# speedup vs baseline: 2.2323x; 2.2323x over previous
import jax
import jax.numpy as jnp
from jax import lax
from jax.experimental import pallas as pl
from jax.experimental.pallas import tpu as pltpu
from jax.experimental.pallas import tpu_sc as plsc

N = 10000
E = 320000
D_IN = 128
D_H = 64
D_OUT = 7
D_OUT_PAD = 16

NC = 2
NS = 16
NW = NC * NS
CHUNK = 100
KSTEPS = 100
EPW = KSTEPS * CHUNK
E_PAD = NW * EPW
N_PAD = 10240
STRIPE = N_PAD // NS

_MESH = plsc.VectorSubcoreMesh(core_axis_name="c", subcore_axis_name="s")
_SC_PARAMS = pltpu.CompilerParams(use_tc_tiling_on_sc=False)


def _deg_body(dst3, zeros, out, idx_v, ones_v, deg_sh):
    c = lax.axis_index("c")
    s = lax.axis_index("s")
    wid = s * NC + c

    @pl.loop(0, CHUNK)
    def _(i):
        ones_v[i, :] = jnp.ones((16,), jnp.float32)

    pltpu.sync_copy(zeros.at[pl.ds(s * STRIPE, STRIPE)],
                    deg_sh.at[pl.ds(s * STRIPE, STRIPE)])
    plsc.subcore_barrier()

    pltpu.sync_copy(dst3.at[wid], idx_v)

    @pl.loop(0, KSTEPS)
    def _(j):
        pltpu.sync_copy(ones_v, deg_sh.at[idx_v.at[j]], add=True)

    plsc.subcore_barrier()
    pltpu.sync_copy(deg_sh.at[pl.ds(s * STRIPE, STRIPE)],
                    out.at[pl.ds(c * N_PAD + s * STRIPE, STRIPE)])


def _deg_counts(dst3, zeros16):
    k = pl.kernel(
        _deg_body,
        out_type=jax.ShapeDtypeStruct((NC * N_PAD, 16), jnp.float32),
        mesh=_MESH,
        scratch_types=[
            pltpu.VMEM((KSTEPS, CHUNK), jnp.int32),
            pltpu.VMEM((CHUNK, 16), jnp.float32),
            pltpu.VMEM_SHARED((N_PAD, 16), jnp.float32),
        ],
        compiler_params=_SC_PARAMS,
    )
    return k(dst3, zeros16)


_DEPTH = 10
_LEAD = 5


def _agg_body(u_hbm, src3, dst3, zeros, out, *scratch):
    isrc, idst = scratch[0], scratch[1]
    rows = list(scratch[2:2 + _DEPTH])
    agg_sh = scratch[2 + _DEPTH]
    gsem = list(scratch[3 + _DEPTH:3 + 2 * _DEPTH])
    ssem = list(scratch[3 + 2 * _DEPTH:3 + 3 * _DEPTH])
    c = lax.axis_index("c")
    s = lax.axis_index("s")
    wid = s * NC + c

    pltpu.sync_copy(zeros.at[pl.ds(s * STRIPE, STRIPE)],
                    agg_sh.at[pl.ds(s * STRIPE, STRIPE)])
    plsc.subcore_barrier()

    pltpu.sync_copy(src3.at[wid], isrc)
    pltpu.sync_copy(dst3.at[wid], idst)

    def wait_gather(b):
        pltpu.make_async_copy(u_hbm.at[isrc.at[0]], rows[b], gsem[b]).wait()

    def wait_scatter(b):
        pltpu.make_async_copy(rows[b], agg_sh.at[idst.at[0]], ssem[b]).wait()

    for i in range(_LEAD):
        pltpu.async_copy(u_hbm.at[isrc.at[i]], rows[i], gsem[i])

    @pl.loop(0, KSTEPS, step=_DEPTH)
    def _(j):
        for o in range(_DEPTH):
            i = j + o
            b = o
            bn = (o + _LEAD) % _DEPTH
            wait_gather(b)
            pltpu.async_copy(rows[b], agg_sh.at[idst.at[i]], ssem[b],
                             add=True)

            @pl.when(i + _LEAD < KSTEPS)
            def _():
                @pl.when(i >= _DEPTH - _LEAD)
                def _():
                    wait_scatter(bn)
                pltpu.async_copy(u_hbm.at[isrc.at[i + _LEAD]], rows[bn],
                                 gsem[bn])

    for b in range(_DEPTH):
        wait_scatter(b)

    plsc.subcore_barrier()
    pltpu.sync_copy(agg_sh.at[pl.ds(s * STRIPE, STRIPE)],
                    out.at[pl.ds(c * N_PAD + s * STRIPE, STRIPE)])


def _edge_aggregate(u, src3, dst3, zeros, width):
    k = pl.kernel(
        _agg_body,
        out_type=jax.ShapeDtypeStruct((NC * N_PAD, width), jnp.float32),
        mesh=_MESH,
        scratch_types=(
            [pltpu.VMEM((KSTEPS, CHUNK), jnp.int32)] * 2
            + [pltpu.VMEM((CHUNK, width), jnp.float32)] * _DEPTH
            + [pltpu.VMEM_SHARED((N_PAD, width), jnp.float32)]
            + [pltpu.SemaphoreType.DMA] * (2 * _DEPTH)
        ),
        compiler_params=_SC_PARAMS,
    )
    return k(u, src3, dst3, zeros)


def _dinv(deg_ref):
    d = deg_ref[0:N, 0:1] + deg_ref[N_PAD:N_PAD + N, 0:1] + 1.0
    return lax.rsqrt(d)


def _mm1_body(x_ref, w1_ref, h_ref):
    h_ref[...] = jnp.dot(x_ref[...], w1_ref[...],
                         preferred_element_type=jnp.float32)


def _u1_body(h_ref, deg_ref, u1_ref):
    u1_ref[...] = _dinv(deg_ref) * h_ref[...]


def _u2_body(agg_ref, u1_ref, deg_ref, b1_ref, w2_ref, u2_ref):
    dinv = _dinv(deg_ref)
    agg = agg_ref[0:N, :] + agg_ref[N_PAD:N_PAD + N, :]
    h1 = jnp.maximum(dinv * (agg + u1_ref[...]) + b1_ref[...], 0.0)
    u2_ref[...] = dinv * jnp.dot(h1, w2_ref[...],
                                 preferred_element_type=jnp.float32)


def _out_body(agg_ref, u2_ref, deg_ref, b2_ref, o_ref):
    dinv = _dinv(deg_ref)
    agg = agg_ref[0:N, :] + agg_ref[N_PAD:N_PAD + N, :]
    o_ref[...] = dinv * (agg + u2_ref[...]) + b2_ref[...]


def _tc_call(body, out_shape):
    return pl.pallas_call(
        body,
        out_shape=jax.ShapeDtypeStruct(out_shape, jnp.float32),
    )


@jax.jit
def kernel(x, edge_index, W1, b1, W2, b2):
    n_pad_e = EPW - E // NW
    src2 = edge_index[0].astype(jnp.int32).reshape(NW, E // NW)
    dst2 = edge_index[1].astype(jnp.int32).reshape(NW, E // NW)
    if n_pad_e:
        pad_src = jnp.zeros((NW, n_pad_e), jnp.int32)
        pad_dst = jnp.broadcast_to(
            N + (jnp.arange(n_pad_e, dtype=jnp.int32) % (N_PAD - N)),
            (NW, n_pad_e))
        src2 = jnp.concatenate([src2, pad_src], axis=1)
        dst2 = jnp.concatenate([dst2, pad_dst], axis=1)
    src3 = src2.reshape(NW, KSTEPS, CHUNK)
    dst3 = dst2.reshape(NW, KSTEPS, CHUNK)
    z16 = jnp.zeros((N_PAD, 16), jnp.float32)
    z64 = jnp.zeros((N_PAD, D_H), jnp.float32)
    w2p = jnp.pad(W2, ((0, 0), (0, D_OUT_PAD - D_OUT)))
    b1r = b1.reshape(1, D_H)
    b2r = jnp.pad(b2, (0, D_OUT_PAD - D_OUT)).reshape(1, D_OUT_PAD)

    deg = _deg_counts(dst3, z16)

    h1 = _tc_call(_mm1_body, (N, D_H))(x, W1)
    u1 = _tc_call(_u1_body, (N, D_H))(h1, deg)
    agg1 = _edge_aggregate(u1, src3, dst3, z64, D_H)
    u2 = _tc_call(_u2_body, (N, D_OUT_PAD))(agg1, u1, deg, b1r, w2p)
    agg2 = _edge_aggregate(u2, src3, dst3, z16, D_OUT_PAD)
    out = _tc_call(_out_body, (N, D_OUT_PAD))(agg2, u2, deg, b2r)
    return out[:, :D_OUT]


# speedup vs baseline: 2.3275x; 1.0426x over previous
import jax
import jax.numpy as jnp
from jax import lax
from jax.experimental import pallas as pl
from jax.experimental.pallas import tpu as pltpu
from jax.experimental.pallas import tpu_sc as plsc

N = 10000
E = 320000
D_IN = 128
D_H = 64
D_OUT = 7
D_OUT_PAD = 16

NC = 2
NS = 16
NW = NC * NS
CHUNK = 100
KSTEPS = 100
EPW = KSTEPS * CHUNK
N_PAD = 10240
STRIPE = N_PAD // NS

assert NW * EPW == E

_MESH = plsc.VectorSubcoreMesh(core_axis_name="c", subcore_axis_name="s")
_SC_PARAMS = pltpu.CompilerParams(use_tc_tiling_on_sc=False)


def _deg_body(ei4, zeros, out, idx_v, ones_v, deg_sh):
    c = lax.axis_index("c")
    s = lax.axis_index("s")
    wid = s * NC + c

    @pl.loop(0, CHUNK)
    def _(i):
        ones_v[i, :] = jnp.ones((16,), jnp.float32)

    pltpu.sync_copy(zeros.at[pl.ds(s * STRIPE, STRIPE)],
                    deg_sh.at[pl.ds(s * STRIPE, STRIPE)])
    plsc.subcore_barrier()

    pltpu.sync_copy(ei4.at[1, wid], idx_v)

    @pl.loop(0, KSTEPS)
    def _(j):
        pltpu.sync_copy(ones_v, deg_sh.at[idx_v.at[j]], add=True)

    plsc.subcore_barrier()
    pltpu.sync_copy(deg_sh.at[pl.ds(s * STRIPE, STRIPE)],
                    out.at[pl.ds(c * N_PAD + s * STRIPE, STRIPE)])


def _deg_counts(ei4, zeros16):
    k = pl.kernel(
        _deg_body,
        out_type=jax.ShapeDtypeStruct((NC * N_PAD, 16), jnp.float32),
        mesh=_MESH,
        scratch_types=[
            pltpu.VMEM((KSTEPS, CHUNK), jnp.int32),
            pltpu.VMEM((CHUNK, 16), jnp.float32),
            pltpu.VMEM_SHARED((N_PAD, 16), jnp.float32),
        ],
        compiler_params=_SC_PARAMS,
    )
    return k(ei4, zeros16)


_DEPTH = 10
_LEAD = 5


def _agg_body(u_hbm, ei4, zeros, out, *scratch):
    isrc, idst = scratch[0], scratch[1]
    rows = list(scratch[2:2 + _DEPTH])
    agg_sh = scratch[2 + _DEPTH]
    gsem = list(scratch[3 + _DEPTH:3 + 2 * _DEPTH])
    ssem = list(scratch[3 + 2 * _DEPTH:3 + 3 * _DEPTH])
    c = lax.axis_index("c")
    s = lax.axis_index("s")
    wid = s * NC + c

    @pl.when(c == 0)
    def _():
        pltpu.sync_copy(u_hbm.at[pl.ds(s * STRIPE, STRIPE)],
                        agg_sh.at[pl.ds(s * STRIPE, STRIPE)])

    @pl.when(c != 0)
    def _():
        pltpu.sync_copy(zeros.at[pl.ds(s * STRIPE, STRIPE)],
                        agg_sh.at[pl.ds(s * STRIPE, STRIPE)])

    plsc.subcore_barrier()

    pltpu.sync_copy(ei4.at[0, wid], isrc)
    pltpu.sync_copy(ei4.at[1, wid], idst)

    def wait_gather(b):
        pltpu.make_async_copy(u_hbm.at[isrc.at[0]], rows[b], gsem[b]).wait()

    def wait_scatter(b):
        pltpu.make_async_copy(rows[b], agg_sh.at[idst.at[0]], ssem[b]).wait()

    for i in range(_LEAD):
        pltpu.async_copy(u_hbm.at[isrc.at[i]], rows[i], gsem[i])

    @pl.loop(0, KSTEPS, step=_DEPTH)
    def _(j):
        for o in range(_DEPTH):
            i = j + o
            b = o
            bn = (o + _LEAD) % _DEPTH
            wait_gather(b)
            pltpu.async_copy(rows[b], agg_sh.at[idst.at[i]], ssem[b],
                             add=True)

            @pl.when(i + _LEAD < KSTEPS)
            def _():
                @pl.when(i >= _DEPTH - _LEAD)
                def _():
                    wait_scatter(bn)
                pltpu.async_copy(u_hbm.at[isrc.at[i + _LEAD]], rows[bn],
                                 gsem[bn])

    for b in range(_DEPTH):
        wait_scatter(b)

    plsc.subcore_barrier()
    pltpu.sync_copy(agg_sh.at[pl.ds(s * STRIPE, STRIPE)],
                    out.at[pl.ds(c * N_PAD + s * STRIPE, STRIPE)])


def _edge_aggregate(u, ei4, zeros, width):
    k = pl.kernel(
        _agg_body,
        out_type=jax.ShapeDtypeStruct((NC * N_PAD, width), jnp.float32),
        mesh=_MESH,
        scratch_types=(
            [pltpu.VMEM((KSTEPS, CHUNK), jnp.int32)] * 2
            + [pltpu.VMEM((CHUNK, width), jnp.float32)] * _DEPTH
            + [pltpu.VMEM_SHARED((N_PAD, width), jnp.float32)]
            + [pltpu.SemaphoreType.DMA] * (2 * _DEPTH)
        ),
        compiler_params=_SC_PARAMS,
    )
    return k(u, ei4, zeros)


def _dinv(deg_ref):
    d = deg_ref[0:N, 0:1] + deg_ref[N_PAD:N_PAD + N, 0:1] + 1.0
    return lax.rsqrt(d)


def _mm1_body(x_ref, w1_ref, h_ref):
    h_ref[...] = jnp.dot(x_ref[...], w1_ref[...],
                         preferred_element_type=jnp.float32)


def _u1_body(h_ref, deg_ref, u1_ref):
    u1_ref[0:N, :] = _dinv(deg_ref) * h_ref[...]
    u1_ref[N:N_PAD, :] = jnp.zeros((N_PAD - N, D_H), jnp.float32)


def _u2_body(agg_ref, deg_ref, b1_ref, w2_ref, u2_ref):
    dinv = _dinv(deg_ref)
    agg = agg_ref[0:N, :] + agg_ref[N_PAD:N_PAD + N, :]
    h1 = jnp.maximum(dinv * agg + b1_ref[...], 0.0)
    u2_ref[0:N, :] = dinv * jnp.dot(h1, w2_ref[...],
                                    preferred_element_type=jnp.float32)
    u2_ref[N:N_PAD, :] = jnp.zeros((N_PAD - N, D_OUT_PAD), jnp.float32)


def _out_body(agg_ref, deg_ref, b2_ref, o_ref):
    agg = agg_ref[0:N, :] + agg_ref[N_PAD:N_PAD + N, :]
    o_ref[...] = _dinv(deg_ref) * agg + b2_ref[...]


def _tc_call(body, out_shape):
    return pl.pallas_call(
        body,
        out_shape=jax.ShapeDtypeStruct(out_shape, jnp.float32),
    )


@jax.jit
def kernel(x, edge_index, W1, b1, W2, b2):
    ei4 = edge_index.astype(jnp.int32).reshape(2, NW, KSTEPS, CHUNK)
    z16 = jnp.zeros((N_PAD, 16), jnp.float32)
    z64 = jnp.zeros((N_PAD, D_H), jnp.float32)
    w2p = jnp.pad(W2, ((0, 0), (0, D_OUT_PAD - D_OUT)))
    b1r = b1.reshape(1, D_H)
    b2r = jnp.pad(b2, (0, D_OUT_PAD - D_OUT)).reshape(1, D_OUT_PAD)

    deg = _deg_counts(ei4, z16)

    h1 = _tc_call(_mm1_body, (N, D_H))(x, W1)
    u1 = _tc_call(_u1_body, (N_PAD, D_H))(h1, deg)
    agg1 = _edge_aggregate(u1, ei4, z64, D_H)
    u2 = _tc_call(_u2_body, (N_PAD, D_OUT_PAD))(agg1, deg, b1r, w2p)
    agg2 = _edge_aggregate(u2, ei4, z16, D_OUT_PAD)
    out = _tc_call(_out_body, (N, D_OUT_PAD))(agg2, deg, b2r)
    return out[:, :D_OUT]
